# Initial kernel scaffold; baseline (speedup 1.0000x reference)
import jax
import jax.numpy as jnp
from jax.experimental import pallas as pl


def kernel(node_attrs, node_feats, edge_attrs, edge_feats, edge_index, magmom_node_inv_feats, magmom_node_attrs, W_up, Wr1, Wr2, Wr3, Wr4, W_density, W_mag_radial, W_lin, W_maglin, W_skip, W_magskip):
    raise NotImplementedError("write your pallas kernel here")


# speedup vs baseline: 2.9685x; 2.9685x over previous
import math

import jax
import jax.numpy as jnp
from jax import lax
from jax.experimental import pallas as pl
from jax.experimental.pallas import tpu as pltpu
from jax.experimental.pallas import tpu_sc as plsc

N = 10000
E = 320000
D = 128
DH = D // 2
NELEM = 10
RADIAL_IN = 8
MAG_IN = 16
H = 64
AVG_NEIGH = 32.0

NSUB = 16
EDGES_PER_TILE = E // NSUB
CHUNK = 80
CHUNKS_PER_TILE = EDGES_PER_TILE // CHUNK
NPAD = 10240
ROWS_PER_TILE = NPAD // NSUB
LAST_ROWS = N - (NSUB - 1) * ROWS_PER_TILE
DEN_W = 16
DEN_ROWS = NPAD // 16
DEN_ROWS_PER_TILE = DEN_ROWS // NSUB

_INV_SQRT_D = 1.0 / math.sqrt(float(D))
_INV_SQRT_R = 1.0 / math.sqrt(float(RADIAL_IN))
_INV_SQRT_H = 1.0 / math.sqrt(float(H))
_INV_SQRT_M = 1.0 / math.sqrt(float(MAG_IN))
_INV_SQRT_UV = 1.0 / math.sqrt(float(D * NELEM))


_NB = 1000


def _pre_body(nf, minv, mga, wup, wmr, xlo, xhi, vlo, vhi):
    x = jnp.dot(nf[...], wup[...], preferred_element_type=jnp.float32) * _INV_SQRT_D
    magr = jnp.dot(minv[...], wmr[...], preferred_element_type=jnp.float32) * _INV_SQRT_M
    v = x * mga[...] * magr
    xlo[...] = x[:, :DH]
    xhi[...] = x[:, DH:]
    vlo[...] = v[:, :DH]
    vhi[...] = v[:, DH:]


def _pre_call(node_feats, minv, mga, W_up, W_mag_radial):
    grid = (N // _NB,)
    return pl.pallas_call(
        _pre_body,
        grid=grid,
        in_specs=[
            pl.BlockSpec((_NB, D), lambda i: (i, 0)),
            pl.BlockSpec((_NB, MAG_IN), lambda i: (i, 0)),
            pl.BlockSpec((_NB, 1), lambda i: (i, 0)),
            pl.BlockSpec((D, D), lambda i: (0, 0)),
            pl.BlockSpec((MAG_IN, D), lambda i: (0, 0)),
        ],
        out_specs=[pl.BlockSpec((_NB, DH), lambda i: (i, 0))] * 4,
        out_shape=[jax.ShapeDtypeStruct((N, DH), jnp.float32)] * 4,
    )(node_feats, minv, mga, W_up, W_mag_radial)


_EB = 2000


def _edge_body(ef, ea, wr1, wr2, wr3, wr4, wd, ewlo, ewhi, eden_out):
    f = ef[...]
    h = jax.nn.silu(jnp.dot(f, wr1[...], preferred_element_type=jnp.float32) * _INV_SQRT_R)
    h = jax.nn.silu(jnp.dot(h, wr2[...], preferred_element_type=jnp.float32) * _INV_SQRT_H)
    h = jax.nn.silu(jnp.dot(h, wr3[...], preferred_element_type=jnp.float32) * _INV_SQRT_H)
    tpw = jnp.dot(h, wr4[...], preferred_element_type=jnp.float32) * _INV_SQRT_H
    ew = tpw * ea[...]
    ewlo[...] = ew[:, :DH]
    ewhi[...] = ew[:, DH:]
    dd = jnp.dot(f, wd[...], preferred_element_type=jnp.float32) * _INV_SQRT_R
    d = jnp.tanh(dd * dd)
    lane = lax.broadcasted_iota(jnp.int32, (_EB, DEN_W), 1)
    eden_out[...] = jnp.where(lane == 0, d, 0.0)


def _edge_call(edge_feats, edge_attrs, Wr1, Wr2, Wr3, Wr4, W_density):
    grid = (E // _EB,)
    return pl.pallas_call(
        _edge_body,
        grid=grid,
        in_specs=[
            pl.BlockSpec((_EB, RADIAL_IN), lambda i: (i, 0)),
            pl.BlockSpec((_EB, 1), lambda i: (i, 0)),
            pl.BlockSpec((RADIAL_IN, H), lambda i: (0, 0)),
            pl.BlockSpec((H, H), lambda i: (0, 0)),
            pl.BlockSpec((H, H), lambda i: (0, 0)),
            pl.BlockSpec((H, D), lambda i: (0, 0)),
            pl.BlockSpec((RADIAL_IN, 1), lambda i: (0, 0)),
        ],
        out_specs=[
            pl.BlockSpec((_EB, DH), lambda i: (i, 0)),
            pl.BlockSpec((_EB, DH), lambda i: (i, 0)),
            pl.BlockSpec((_EB, DEN_W), lambda i: (i, 0)),
        ],
        out_shape=[
            jax.ShapeDtypeStruct((E, DH), jnp.float32),
            jax.ShapeDtypeStruct((E, DH), jnp.float32),
            jax.ShapeDtypeStruct((E, DEN_W), jnp.float32),
        ],
    )(edge_feats, edge_attrs, Wr1, Wr2, Wr3, Wr4, W_density)


def _zero_rows(rows):
    zero16 = jnp.zeros((16,), jnp.float32)

    def _z(i, _):
        for j in range(DH // 16):
            rows[i, pl.ds(j * 16, 16)] = zero16
        return 0

    lax.fori_loop(0, CHUNK, _z, 0)


def _zero_acc_slice(rows, shared_acc, sid):
    base = sid * ROWS_PER_TILE
    for z in range(ROWS_PER_TILE // CHUNK):
        pltpu.sync_copy(rows, shared_acc.at[pl.ds(base + z * CHUNK, CHUNK)])


def _gather_pass(src, ewsrc, eden, sidx, ridx, rows, ewb, edenb,
                 shared_acc, shared_den, gsem, esem, dsem, sid, do_mul, do_den):
    def _chunk(c, _):
        ebase = sid * EDGES_PER_TILE + c * CHUNK
        ga = pltpu.async_copy(src.at[sidx.at[c]], rows, gsem)
        if do_mul:
            ea = pltpu.async_copy(ewsrc.at[pl.ds(ebase, CHUNK)], ewb, esem)
        if do_den:
            da = pltpu.async_copy(eden.at[pl.ds(ebase, CHUNK)], edenb, dsem)
        ga.wait()
        if do_mul:
            ea.wait()

            def _mul(i, _):
                for j in range(DH // 16):
                    sl = pl.ds(j * 16, 16)
                    rows[i, sl] = rows[i, sl] * ewb[i, sl]
                return 0

            lax.fori_loop(0, CHUNK, _mul, 0)
        pltpu.sync_copy(rows, shared_acc.at[ridx.at[c]], add=True)
        if do_den:
            da.wait()
            pltpu.sync_copy(edenb, shared_den.at[ridx.at[c]], add=True)
        return 0

    lax.fori_loop(0, CHUNKS_PER_TILE, _chunk, 0)


def _copy_out(shared_acc, out, sid):
    @pl.when(sid < NSUB - 1)
    def _full():
        osl = pl.ds(sid * ROWS_PER_TILE, ROWS_PER_TILE)
        pltpu.sync_copy(shared_acc.at[osl], out.at[osl])

    @pl.when(sid == NSUB - 1)
    def _part():
        osl = pl.ds((NSUB - 1) * ROWS_PER_TILE, LAST_ROWS)
        pltpu.sync_copy(shared_acc.at[osl], out.at[osl])


def _sc_body(s3d, r3d, xlo, xhi, vlo, vhi, ewlo, ewhi, eden,
             msg_lo, msg_hi, mag_lo, mag_hi, den_out,
             sidx, ridx, rows, ewb, edenb,
             shared_acc, shared_den,
             gsem, esem, dsem):
    cid = lax.axis_index("c")
    sid = lax.axis_index("s")

    pltpu.sync_copy(s3d.at[sid], sidx)
    pltpu.sync_copy(r3d.at[sid], ridx)

    _zero_rows(rows)
    zero16 = jnp.zeros((16,), jnp.float32)

    def _zd(i, _):
        edenb[i, pl.ds(0, 16)] = zero16
        return 0

    lax.fori_loop(0, CHUNK, _zd, 0)
    _zero_acc_slice(rows, shared_acc, sid)
    base = sid * ROWS_PER_TILE
    for z in range(ROWS_PER_TILE // CHUNK):
        pltpu.sync_copy(edenb, shared_den.at[pl.ds(base + z * CHUNK, CHUNK)])
    plsc.subcore_barrier()

    @pl.when(cid == 0)
    def _a0():
        _gather_pass(xlo, ewlo, eden, sidx, ridx, rows, ewb, edenb,
                     shared_acc, shared_den, gsem, esem, dsem, sid, True, True)

    @pl.when(cid == 1)
    def _a1():
        _gather_pass(xhi, ewhi, eden, sidx, ridx, rows, ewb, edenb,
                     shared_acc, shared_den, gsem, esem, dsem, sid, True, False)

    plsc.subcore_barrier()

    @pl.when(cid == 0)
    def _oa0():
        _copy_out(shared_acc, msg_lo, sid)
        dsl = pl.ds(sid * ROWS_PER_TILE, ROWS_PER_TILE)
        pltpu.sync_copy(shared_den.at[dsl], den_out.at[dsl])

    @pl.when(cid == 1)
    def _oa1():
        _copy_out(shared_acc, msg_hi, sid)

    plsc.subcore_barrier()

    _zero_rows(rows)
    _zero_acc_slice(rows, shared_acc, sid)
    plsc.subcore_barrier()

    @pl.when(cid == 0)
    def _b0():
        _gather_pass(vlo, ewlo, eden, sidx, ridx, rows, ewb, edenb,
                     shared_acc, shared_den, gsem, esem, dsem, sid, False, False)

    @pl.when(cid == 1)
    def _b1():
        _gather_pass(vhi, ewhi, eden, sidx, ridx, rows, ewb, edenb,
                     shared_acc, shared_den, gsem, esem, dsem, sid, False, False)

    plsc.subcore_barrier()

    @pl.when(cid == 0)
    def _ob0():
        _copy_out(shared_acc, mag_lo, sid)

    @pl.when(cid == 1)
    def _ob1():
        _copy_out(shared_acc, mag_hi, sid)


def _sc_call(s3d, r3d, xlo, xhi, vlo, vhi, ewlo, ewhi, eden):
    mesh = plsc.VectorSubcoreMesh(core_axis_name="c", subcore_axis_name="s")
    f = pl.kernel(
        _sc_body,
        compiler_params=pltpu.CompilerParams(use_tc_tiling_on_sc=False),
        out_type=[
            jax.ShapeDtypeStruct((N, DH), jnp.float32),
            jax.ShapeDtypeStruct((N, DH), jnp.float32),
            jax.ShapeDtypeStruct((N, DH), jnp.float32),
            jax.ShapeDtypeStruct((N, DH), jnp.float32),
            jax.ShapeDtypeStruct((NPAD, DEN_W), jnp.float32),
        ],
        mesh=mesh,
        scratch_types=[
            pltpu.VMEM((CHUNKS_PER_TILE, CHUNK), jnp.int32),
            pltpu.VMEM((CHUNKS_PER_TILE, CHUNK), jnp.int32),
            pltpu.VMEM((CHUNK, DH), jnp.float32),
            pltpu.VMEM((CHUNK, DH), jnp.float32),
            pltpu.VMEM((CHUNK, DEN_W), jnp.float32),
            pltpu.VMEM_SHARED((NPAD, DH), jnp.float32),
            pltpu.VMEM_SHARED((NPAD, DEN_W), jnp.float32),
            pltpu.SemaphoreType.DMA,
            pltpu.SemaphoreType.DMA,
            pltpu.SemaphoreType.DMA,
        ],
    )
    return f(s3d, r3d, xlo, xhi, vlo, vhi, ewlo, ewhi, eden)


def _post_body(mlo, mhi, glo, ghi, den, na, wl_lo, wl_hi, wm_lo, wm_hi,
               wskip, wmagskip, o1, o2):
    m = (jnp.dot(mlo[...], wl_lo[...], preferred_element_type=jnp.float32)
         + jnp.dot(mhi[...], wl_hi[...], preferred_element_type=jnp.float32)) * _INV_SQRT_D
    m = m / (den[...] + 1.0)
    g = (jnp.dot(glo[...], wm_lo[...], preferred_element_type=jnp.float32)
         + jnp.dot(ghi[...], wm_hi[...], preferred_element_type=jnp.float32)) * (
        _INV_SQRT_D / AVG_NEIGH)
    a = na[...]
    acc1 = jnp.zeros((_NB, D), jnp.float32)
    acc2 = jnp.zeros((_NB, D), jnp.float32)
    for vv in range(NELEM):
        av = a[:, vv:vv + 1]
        acc1 = acc1 + jnp.dot(m * av, wskip[vv], preferred_element_type=jnp.float32)
        acc2 = acc2 + jnp.dot(g * av, wmagskip[vv], preferred_element_type=jnp.float32)
    o1[...] = acc1 * _INV_SQRT_UV
    o2[...] = acc2 * _INV_SQRT_UV


def _post_call(mlo, mhi, glo, ghi, den2d, node_attrs,
               wl_lo, wl_hi, wm_lo, wm_hi, W_skip_t, W_magskip_t):
    grid = (N // _NB,)
    return pl.pallas_call(
        _post_body,
        grid=grid,
        in_specs=[
            pl.BlockSpec((_NB, DH), lambda i: (i, 0)),
            pl.BlockSpec((_NB, DH), lambda i: (i, 0)),
            pl.BlockSpec((_NB, DH), lambda i: (i, 0)),
            pl.BlockSpec((_NB, DH), lambda i: (i, 0)),
            pl.BlockSpec((_NB, 1), lambda i: (i, 0)),
            pl.BlockSpec((_NB, NELEM), lambda i: (i, 0)),
            pl.BlockSpec((DH, D), lambda i: (0, 0)),
            pl.BlockSpec((DH, D), lambda i: (0, 0)),
            pl.BlockSpec((DH, D), lambda i: (0, 0)),
            pl.BlockSpec((DH, D), lambda i: (0, 0)),
            pl.BlockSpec((NELEM, D, D), lambda i: (0, 0, 0)),
            pl.BlockSpec((NELEM, D, D), lambda i: (0, 0, 0)),
        ],
        out_specs=[
            pl.BlockSpec((_NB, D), lambda i: (i, 0)),
            pl.BlockSpec((_NB, D), lambda i: (i, 0)),
        ],
        out_shape=[
            jax.ShapeDtypeStruct((N, D), jnp.float32),
            jax.ShapeDtypeStruct((N, D), jnp.float32),
        ],
    )(mlo, mhi, glo, ghi, den2d, node_attrs, wl_lo, wl_hi, wm_lo, wm_hi,
      W_skip_t, W_magskip_t)


def kernel(node_attrs, node_feats, edge_attrs, edge_feats, edge_index,
           magmom_node_inv_feats, magmom_node_attrs,
           W_up, Wr1, Wr2, Wr3, Wr4, W_density, W_mag_radial,
           W_lin, W_maglin, W_skip, W_magskip):
    xlo, xhi, vlo, vhi = _pre_call(node_feats, magmom_node_inv_feats,
                                   magmom_node_attrs, W_up, W_mag_radial)
    ewlo, ewhi, eden = _edge_call(edge_feats, edge_attrs, Wr1, Wr2, Wr3, Wr4,
                                  W_density)
    s3d = edge_index[0].reshape(NSUB, CHUNKS_PER_TILE, CHUNK)
    r3d = edge_index[1].reshape(NSUB, CHUNKS_PER_TILE, CHUNK)
    msg_lo, msg_hi, mag_lo, mag_hi, den = _sc_call(
        s3d, r3d, xlo, xhi, vlo, vhi, ewlo, ewhi, eden)
    W_skip_t = jnp.transpose(W_skip, (1, 0, 2))
    W_magskip_t = jnp.transpose(W_magskip, (1, 0, 2))
    den1 = den[:N, 0:1]
    o1, o2 = _post_call(msg_lo, msg_hi, mag_lo, mag_hi, den1,
                        node_attrs, W_lin[:DH], W_lin[DH:],
                        W_maglin[:DH], W_maglin[DH:], W_skip_t, W_magskip_t)
    return (o1.reshape(N, D, 1), o2.reshape(N, D, 1))


# speedup vs baseline: 3.6980x; 1.2457x over previous
import math

import jax
import jax.numpy as jnp
from jax import lax
from jax.experimental import pallas as pl
from jax.experimental.pallas import tpu as pltpu
from jax.experimental.pallas import tpu_sc as plsc

N = 10000
E = 320000
D = 128
DH = D // 2
NELEM = 10
RADIAL_IN = 8
MAG_IN = 16
H = 64
AVG_NEIGH = 32.0

NSUB = 16
EDGES_PER_TILE = E // NSUB
CHUNK = 80
CHUNKS_PER_TILE = EDGES_PER_TILE // CHUNK
NPAD = 10240
ROWS_PER_TILE = NPAD // NSUB
LAST_ROWS = N - (NSUB - 1) * ROWS_PER_TILE
DEN_W = 16
DEN_ROWS = NPAD // 16
DEN_ROWS_PER_TILE = DEN_ROWS // NSUB

_INV_SQRT_D = 1.0 / math.sqrt(float(D))
_INV_SQRT_R = 1.0 / math.sqrt(float(RADIAL_IN))
_INV_SQRT_H = 1.0 / math.sqrt(float(H))
_INV_SQRT_M = 1.0 / math.sqrt(float(MAG_IN))
_INV_SQRT_UV = 1.0 / math.sqrt(float(D * NELEM))


_NB = 1000


def _pre_body(nf, minv, mga, wup, wmr, xlo, xhi, vlo, vhi):
    x = jnp.dot(nf[...], wup[...], preferred_element_type=jnp.float32) * _INV_SQRT_D
    magr = jnp.dot(minv[...], wmr[...], preferred_element_type=jnp.float32) * _INV_SQRT_M
    v = x * mga[...] * magr
    xlo[...] = x[:, :DH]
    xhi[...] = x[:, DH:]
    vlo[...] = v[:, :DH]
    vhi[...] = v[:, DH:]


def _pre_call(node_feats, minv, mga, W_up, W_mag_radial):
    grid = (N // _NB,)
    return pl.pallas_call(
        _pre_body,
        grid=grid,
        in_specs=[
            pl.BlockSpec((_NB, D), lambda i: (i, 0)),
            pl.BlockSpec((_NB, MAG_IN), lambda i: (i, 0)),
            pl.BlockSpec((_NB, 1), lambda i: (i, 0)),
            pl.BlockSpec((D, D), lambda i: (0, 0)),
            pl.BlockSpec((MAG_IN, D), lambda i: (0, 0)),
        ],
        out_specs=[pl.BlockSpec((_NB, DH), lambda i: (i, 0))] * 4,
        out_shape=[jax.ShapeDtypeStruct((N, DH), jnp.float32)] * 4,
    )(node_feats, minv, mga, W_up, W_mag_radial)


_EB = 2000


def _edge_body(ef, ea, wr1, wr2, wr3, wr4, wd, ewlo, ewhi, eden_out):
    f = ef[...]
    h = jax.nn.silu(jnp.dot(f, wr1[...], preferred_element_type=jnp.float32) * _INV_SQRT_R)
    h = jax.nn.silu(jnp.dot(h, wr2[...], preferred_element_type=jnp.float32) * _INV_SQRT_H)
    h = jax.nn.silu(jnp.dot(h, wr3[...], preferred_element_type=jnp.float32) * _INV_SQRT_H)
    tpw = jnp.dot(h, wr4[...], preferred_element_type=jnp.float32) * _INV_SQRT_H
    ew = tpw * ea[...]
    ewlo[...] = ew[:, :DH]
    ewhi[...] = ew[:, DH:]
    dd = jnp.dot(f, wd[...], preferred_element_type=jnp.float32) * _INV_SQRT_R
    d = jnp.tanh(dd * dd)
    lane = lax.broadcasted_iota(jnp.int32, (_EB, DEN_W), 1)
    eden_out[...] = jnp.where(lane == 0, d, 0.0)


def _edge_call(edge_feats, edge_attrs, Wr1, Wr2, Wr3, Wr4, W_density):
    grid = (E // _EB,)
    return pl.pallas_call(
        _edge_body,
        grid=grid,
        in_specs=[
            pl.BlockSpec((_EB, RADIAL_IN), lambda i: (i, 0)),
            pl.BlockSpec((_EB, 1), lambda i: (i, 0)),
            pl.BlockSpec((RADIAL_IN, H), lambda i: (0, 0)),
            pl.BlockSpec((H, H), lambda i: (0, 0)),
            pl.BlockSpec((H, H), lambda i: (0, 0)),
            pl.BlockSpec((H, D), lambda i: (0, 0)),
            pl.BlockSpec((RADIAL_IN, 1), lambda i: (0, 0)),
        ],
        out_specs=[
            pl.BlockSpec((_EB, DH), lambda i: (i, 0)),
            pl.BlockSpec((_EB, DH), lambda i: (i, 0)),
            pl.BlockSpec((_EB, DEN_W), lambda i: (i, 0)),
        ],
        out_shape=[
            jax.ShapeDtypeStruct((E, DH), jnp.float32),
            jax.ShapeDtypeStruct((E, DH), jnp.float32),
            jax.ShapeDtypeStruct((E, DEN_W), jnp.float32),
        ],
    )(edge_feats, edge_attrs, Wr1, Wr2, Wr3, Wr4, W_density)


def _zero_rows(rows):
    zero16 = jnp.zeros((16,), jnp.float32)

    def _z(i, _):
        for j in range(DH // 16):
            rows[0, i, pl.ds(j * 16, 16)] = zero16
        return 0

    lax.fori_loop(0, CHUNK, _z, 0)


def _zero_acc_slice(rows, shared_acc, sid):
    base = sid * ROWS_PER_TILE
    for z in range(ROWS_PER_TILE // CHUNK):
        pltpu.sync_copy(rows.at[0], shared_acc.at[pl.ds(base + z * CHUNK, CHUNK)])


def _gather_pass(src, ewsrc, eden, sidx, ridx, rows, ewb, edenb,
                 shared_acc, shared_den, gsems, esems, dsems, sid,
                 do_mul, do_den):
    ebase0 = sid * EDGES_PER_TILE

    def _issue(c, b):
        pltpu.async_copy(src.at[sidx.at[c]], rows.at[b], gsems[b])
        if do_mul:
            pltpu.async_copy(ewsrc.at[pl.ds(ebase0 + c * CHUNK, CHUNK)],
                             ewb.at[b], esems[b])
        if do_den:
            pltpu.async_copy(eden.at[pl.ds(ebase0 + c * CHUNK, CHUNK)],
                             edenb.at[b], dsems[b])

    def _wait(c, b):
        pltpu.make_async_copy(src.at[sidx.at[c]], rows.at[b], gsems[b]).wait()
        if do_mul:
            pltpu.make_async_copy(ewsrc.at[pl.ds(ebase0 + c * CHUNK, CHUNK)],
                                  ewb.at[b], esems[b]).wait()
        if do_den:
            pltpu.make_async_copy(eden.at[pl.ds(ebase0 + c * CHUNK, CHUNK)],
                                  edenb.at[b], dsems[b]).wait()

    _issue(0, 0)

    def _chunk2(c2, _):
        for b in range(2):
            c = c2 * 2 + b

            @pl.when(c < CHUNKS_PER_TILE - 1)
            def _pref():
                _issue(c + 1, 1 - b)

            _wait(c, b)
            if do_mul:
                def _mul(i, _2):
                    for j in range(DH // 16):
                        sl = pl.ds(j * 16, 16)
                        rows[b, i, sl] = rows[b, i, sl] * ewb[b, i, sl]
                    return 0

                lax.fori_loop(0, CHUNK, _mul, 0)
            pltpu.sync_copy(rows.at[b], shared_acc.at[ridx.at[c]], add=True)
            if do_den:
                pltpu.sync_copy(edenb.at[b], shared_den.at[ridx.at[c]], add=True)
        return 0

    lax.fori_loop(0, CHUNKS_PER_TILE // 2, _chunk2, 0)


def _copy_out(shared_acc, out, sid):
    @pl.when(sid < NSUB - 1)
    def _full():
        osl = pl.ds(sid * ROWS_PER_TILE, ROWS_PER_TILE)
        pltpu.sync_copy(shared_acc.at[osl], out.at[osl])

    @pl.when(sid == NSUB - 1)
    def _part():
        osl = pl.ds((NSUB - 1) * ROWS_PER_TILE, LAST_ROWS)
        pltpu.sync_copy(shared_acc.at[osl], out.at[osl])


def _sc_body(s3d, r3d, xlo, xhi, vlo, vhi, ewlo, ewhi, eden,
             msg_lo, msg_hi, mag_lo, mag_hi, den_out,
             sidx, ridx, rows, ewb, edenb,
             shared_acc, shared_den,
             gsem0, gsem1, esem0, esem1, dsem0, dsem1):
    gsems = (gsem0, gsem1)
    esems = (esem0, esem1)
    dsems = (dsem0, dsem1)
    cid = lax.axis_index("c")
    sid = lax.axis_index("s")

    pltpu.sync_copy(s3d.at[sid], sidx)
    pltpu.sync_copy(r3d.at[sid], ridx)

    _zero_rows(rows)
    zero16 = jnp.zeros((16,), jnp.float32)

    def _zd(i, _):
        edenb[0, i, pl.ds(0, 16)] = zero16
        return 0

    lax.fori_loop(0, CHUNK, _zd, 0)
    _zero_acc_slice(rows, shared_acc, sid)
    base = sid * ROWS_PER_TILE
    for z in range(ROWS_PER_TILE // CHUNK):
        pltpu.sync_copy(edenb.at[0], shared_den.at[pl.ds(base + z * CHUNK, CHUNK)])
    plsc.subcore_barrier()

    @pl.when(cid == 0)
    def _a0():
        _gather_pass(xlo, ewlo, eden, sidx, ridx, rows, ewb, edenb,
                     shared_acc, shared_den, gsems, esems, dsems, sid, True, True)

    @pl.when(cid == 1)
    def _a1():
        _gather_pass(xhi, ewhi, eden, sidx, ridx, rows, ewb, edenb,
                     shared_acc, shared_den, gsems, esems, dsems, sid, True, False)

    plsc.subcore_barrier()

    @pl.when(cid == 0)
    def _oa0():
        _copy_out(shared_acc, msg_lo, sid)
        dsl = pl.ds(sid * ROWS_PER_TILE, ROWS_PER_TILE)
        pltpu.sync_copy(shared_den.at[dsl], den_out.at[dsl])

    @pl.when(cid == 1)
    def _oa1():
        _copy_out(shared_acc, msg_hi, sid)

    plsc.subcore_barrier()

    _zero_rows(rows)
    _zero_acc_slice(rows, shared_acc, sid)
    plsc.subcore_barrier()

    @pl.when(cid == 0)
    def _b0():
        _gather_pass(vlo, ewlo, eden, sidx, ridx, rows, ewb, edenb,
                     shared_acc, shared_den, gsems, esems, dsems, sid, False, False)

    @pl.when(cid == 1)
    def _b1():
        _gather_pass(vhi, ewhi, eden, sidx, ridx, rows, ewb, edenb,
                     shared_acc, shared_den, gsems, esems, dsems, sid, False, False)

    plsc.subcore_barrier()

    @pl.when(cid == 0)
    def _ob0():
        _copy_out(shared_acc, mag_lo, sid)

    @pl.when(cid == 1)
    def _ob1():
        _copy_out(shared_acc, mag_hi, sid)


def _sc_call(s3d, r3d, xlo, xhi, vlo, vhi, ewlo, ewhi, eden):
    mesh = plsc.VectorSubcoreMesh(core_axis_name="c", subcore_axis_name="s")
    f = pl.kernel(
        _sc_body,
        compiler_params=pltpu.CompilerParams(use_tc_tiling_on_sc=False),
        out_type=[
            jax.ShapeDtypeStruct((N, DH), jnp.float32),
            jax.ShapeDtypeStruct((N, DH), jnp.float32),
            jax.ShapeDtypeStruct((N, DH), jnp.float32),
            jax.ShapeDtypeStruct((N, DH), jnp.float32),
            jax.ShapeDtypeStruct((NPAD, DEN_W), jnp.float32),
        ],
        mesh=mesh,
        scratch_types=[
            pltpu.VMEM((CHUNKS_PER_TILE, CHUNK), jnp.int32),
            pltpu.VMEM((CHUNKS_PER_TILE, CHUNK), jnp.int32),
            pltpu.VMEM((2, CHUNK, DH), jnp.float32),
            pltpu.VMEM((2, CHUNK, DH), jnp.float32),
            pltpu.VMEM((2, CHUNK, DEN_W), jnp.float32),
            pltpu.VMEM_SHARED((NPAD, DH), jnp.float32),
            pltpu.VMEM_SHARED((NPAD, DEN_W), jnp.float32),
            pltpu.SemaphoreType.DMA,
            pltpu.SemaphoreType.DMA,
            pltpu.SemaphoreType.DMA,
            pltpu.SemaphoreType.DMA,
            pltpu.SemaphoreType.DMA,
            pltpu.SemaphoreType.DMA,
        ],
    )
    return f(s3d, r3d, xlo, xhi, vlo, vhi, ewlo, ewhi, eden)


def _post_body(mlo, mhi, glo, ghi, den, na, wl_lo, wl_hi, wm_lo, wm_hi,
               wskip, wmagskip, o1, o2):
    m = (jnp.dot(mlo[...], wl_lo[...], preferred_element_type=jnp.float32)
         + jnp.dot(mhi[...], wl_hi[...], preferred_element_type=jnp.float32)) * _INV_SQRT_D
    m = m / (den[...] + 1.0)
    g = (jnp.dot(glo[...], wm_lo[...], preferred_element_type=jnp.float32)
         + jnp.dot(ghi[...], wm_hi[...], preferred_element_type=jnp.float32)) * (
        _INV_SQRT_D / AVG_NEIGH)
    a = na[...]
    acc1 = jnp.zeros((_NB, D), jnp.float32)
    acc2 = jnp.zeros((_NB, D), jnp.float32)
    for vv in range(NELEM):
        av = a[:, vv:vv + 1]
        acc1 = acc1 + jnp.dot(m * av, wskip[vv], preferred_element_type=jnp.float32)
        acc2 = acc2 + jnp.dot(g * av, wmagskip[vv], preferred_element_type=jnp.float32)
    o1[...] = acc1 * _INV_SQRT_UV
    o2[...] = acc2 * _INV_SQRT_UV


def _post_call(mlo, mhi, glo, ghi, den2d, node_attrs,
               wl_lo, wl_hi, wm_lo, wm_hi, W_skip_t, W_magskip_t):
    grid = (N // _NB,)
    return pl.pallas_call(
        _post_body,
        grid=grid,
        in_specs=[
            pl.BlockSpec((_NB, DH), lambda i: (i, 0)),
            pl.BlockSpec((_NB, DH), lambda i: (i, 0)),
            pl.BlockSpec((_NB, DH), lambda i: (i, 0)),
            pl.BlockSpec((_NB, DH), lambda i: (i, 0)),
            pl.BlockSpec((_NB, 1), lambda i: (i, 0)),
            pl.BlockSpec((_NB, NELEM), lambda i: (i, 0)),
            pl.BlockSpec((DH, D), lambda i: (0, 0)),
            pl.BlockSpec((DH, D), lambda i: (0, 0)),
            pl.BlockSpec((DH, D), lambda i: (0, 0)),
            pl.BlockSpec((DH, D), lambda i: (0, 0)),
            pl.BlockSpec((NELEM, D, D), lambda i: (0, 0, 0)),
            pl.BlockSpec((NELEM, D, D), lambda i: (0, 0, 0)),
        ],
        out_specs=[
            pl.BlockSpec((_NB, D), lambda i: (i, 0)),
            pl.BlockSpec((_NB, D), lambda i: (i, 0)),
        ],
        out_shape=[
            jax.ShapeDtypeStruct((N, D), jnp.float32),
            jax.ShapeDtypeStruct((N, D), jnp.float32),
        ],
    )(mlo, mhi, glo, ghi, den2d, node_attrs, wl_lo, wl_hi, wm_lo, wm_hi,
      W_skip_t, W_magskip_t)


def kernel(node_attrs, node_feats, edge_attrs, edge_feats, edge_index,
           magmom_node_inv_feats, magmom_node_attrs,
           W_up, Wr1, Wr2, Wr3, Wr4, W_density, W_mag_radial,
           W_lin, W_maglin, W_skip, W_magskip):
    xlo, xhi, vlo, vhi = _pre_call(node_feats, magmom_node_inv_feats,
                                   magmom_node_attrs, W_up, W_mag_radial)
    ewlo, ewhi, eden = _edge_call(edge_feats, edge_attrs, Wr1, Wr2, Wr3, Wr4,
                                  W_density)
    s3d = edge_index[0].reshape(NSUB, CHUNKS_PER_TILE, CHUNK)
    r3d = edge_index[1].reshape(NSUB, CHUNKS_PER_TILE, CHUNK)
    msg_lo, msg_hi, mag_lo, mag_hi, den = _sc_call(
        s3d, r3d, xlo, xhi, vlo, vhi, ewlo, ewhi, eden)
    W_skip_t = jnp.transpose(W_skip, (1, 0, 2))
    W_magskip_t = jnp.transpose(W_magskip, (1, 0, 2))
    den1 = den[:N, 0:1]
    o1, o2 = _post_call(msg_lo, msg_hi, mag_lo, mag_hi, den1,
                        node_attrs, W_lin[:DH], W_lin[DH:],
                        W_maglin[:DH], W_maglin[DH:], W_skip_t, W_magskip_t)
    return (o1.reshape(N, D, 1), o2.reshape(N, D, 1))


# speedup vs baseline: 3.8676x; 1.0458x over previous
import math

import jax
import jax.numpy as jnp
from jax import lax
from jax.experimental import pallas as pl
from jax.experimental.pallas import tpu as pltpu
from jax.experimental.pallas import tpu_sc as plsc

N = 10000
E = 320000
D = 128
DH = D // 2
NELEM = 10
RADIAL_IN = 8
MAG_IN = 16
H = 64
AVG_NEIGH = 32.0

NSUB = 16
EDGES_PER_TILE = E // NSUB
CHUNK = 80
CHUNKS_PER_TILE = EDGES_PER_TILE // CHUNK
NPAD = 10240
ROWS_PER_TILE = NPAD // NSUB
LAST_ROWS = N - (NSUB - 1) * ROWS_PER_TILE
DEN_W = 16
DEN_ROWS = NPAD // 16
DEN_ROWS_PER_TILE = DEN_ROWS // NSUB

_INV_SQRT_D = 1.0 / math.sqrt(float(D))
_INV_SQRT_R = 1.0 / math.sqrt(float(RADIAL_IN))
_INV_SQRT_H = 1.0 / math.sqrt(float(H))
_INV_SQRT_M = 1.0 / math.sqrt(float(MAG_IN))
_INV_SQRT_UV = 1.0 / math.sqrt(float(D * NELEM))


_NB = 1000


def _pre_body(nf, minv, mga, wup, wmr, xlo, xhi, vlo, vhi):
    x = jnp.dot(nf[...], wup[...], preferred_element_type=jnp.float32) * _INV_SQRT_D
    magr = jnp.dot(minv[...], wmr[...], preferred_element_type=jnp.float32) * _INV_SQRT_M
    v = x * mga[...] * magr
    xlo[...] = x[:, :DH]
    xhi[...] = x[:, DH:]
    vlo[...] = v[:, :DH]
    vhi[...] = v[:, DH:]


def _pre_call(node_feats, minv, mga, W_up, W_mag_radial):
    grid = (N // _NB,)
    return pl.pallas_call(
        _pre_body,
        grid=grid,
        in_specs=[
            pl.BlockSpec((_NB, D), lambda i: (i, 0)),
            pl.BlockSpec((_NB, MAG_IN), lambda i: (i, 0)),
            pl.BlockSpec((_NB, 1), lambda i: (i, 0)),
            pl.BlockSpec((D, D), lambda i: (0, 0)),
            pl.BlockSpec((MAG_IN, D), lambda i: (0, 0)),
        ],
        out_specs=[pl.BlockSpec((_NB, DH), lambda i: (i, 0))] * 4,
        out_shape=[jax.ShapeDtypeStruct((N, DH), jnp.float32)] * 4,
    )(node_feats, minv, mga, W_up, W_mag_radial)


_EB = 2000


def _edge_body(ef, ea, wr1, wr2, wr3, wr4, wd, ewlo, ewhi, eden_out):
    f = ef[...]
    h = jax.nn.silu(jnp.dot(f, wr1[...], preferred_element_type=jnp.float32) * _INV_SQRT_R)
    h = jax.nn.silu(jnp.dot(h, wr2[...], preferred_element_type=jnp.float32) * _INV_SQRT_H)
    h = jax.nn.silu(jnp.dot(h, wr3[...], preferred_element_type=jnp.float32) * _INV_SQRT_H)
    tpw = jnp.dot(h, wr4[...], preferred_element_type=jnp.float32) * _INV_SQRT_H
    ew = tpw * ea[...]
    ewlo[...] = ew[:, :DH]
    ewhi[...] = ew[:, DH:]
    dd = jnp.dot(f, wd[...], preferred_element_type=jnp.float32) * _INV_SQRT_R
    d = jnp.tanh(dd * dd)
    lane = lax.broadcasted_iota(jnp.int32, (_EB, DEN_W), 1)
    eden_out[...] = jnp.where(lane == 0, d, 0.0)


def _edge_call(edge_feats, edge_attrs, Wr1, Wr2, Wr3, Wr4, W_density):
    grid = (E // _EB,)
    return pl.pallas_call(
        _edge_body,
        grid=grid,
        in_specs=[
            pl.BlockSpec((_EB, RADIAL_IN), lambda i: (i, 0)),
            pl.BlockSpec((_EB, 1), lambda i: (i, 0)),
            pl.BlockSpec((RADIAL_IN, H), lambda i: (0, 0)),
            pl.BlockSpec((H, H), lambda i: (0, 0)),
            pl.BlockSpec((H, H), lambda i: (0, 0)),
            pl.BlockSpec((H, D), lambda i: (0, 0)),
            pl.BlockSpec((RADIAL_IN, 1), lambda i: (0, 0)),
        ],
        out_specs=[
            pl.BlockSpec((_EB, DH), lambda i: (i, 0)),
            pl.BlockSpec((_EB, DH), lambda i: (i, 0)),
            pl.BlockSpec((_EB, DEN_W), lambda i: (i, 0)),
        ],
        out_shape=[
            jax.ShapeDtypeStruct((E, DH), jnp.float32),
            jax.ShapeDtypeStruct((E, DH), jnp.float32),
            jax.ShapeDtypeStruct((E, DEN_W), jnp.float32),
        ],
    )(edge_feats, edge_attrs, Wr1, Wr2, Wr3, Wr4, W_density)


def _zero_rows(rows):
    zero16 = jnp.zeros((16,), jnp.float32)

    def _z(i, _):
        for j in range(DH // 16):
            rows[0, i, pl.ds(j * 16, 16)] = zero16
        return 0

    lax.fori_loop(0, CHUNK, _z, 0)


def _zero_acc_slice(rows, shared_acc, sid):
    base = sid * ROWS_PER_TILE
    for z in range(ROWS_PER_TILE // CHUNK):
        pltpu.sync_copy(rows.at[0], shared_acc.at[pl.ds(base + z * CHUNK, CHUNK)])


def _gather_pass(src, ewsrc, eden, sidx, ridx, rows, ewb, edenb,
                 shared_acc, shared_den, gsems, esems, dsems, ssems, tsems,
                 sid, do_mul, do_den):
    ebase0 = sid * EDGES_PER_TILE

    def _issue(c, s):
        pltpu.async_copy(src.at[sidx.at[c]], rows.at[s], gsems[s])
        if do_mul:
            pltpu.async_copy(ewsrc.at[pl.ds(ebase0 + c * CHUNK, CHUNK)],
                             ewb.at[s], esems[s])
        if do_den:
            pltpu.async_copy(eden.at[pl.ds(ebase0 + c * CHUNK, CHUNK)],
                             edenb.at[s], dsems[s])

    def _wait_in(c, s):
        pltpu.make_async_copy(src.at[sidx.at[c]], rows.at[s], gsems[s]).wait()
        if do_mul:
            pltpu.make_async_copy(ewsrc.at[pl.ds(ebase0 + c * CHUNK, CHUNK)],
                                  ewb.at[s], esems[s]).wait()
        if do_den:
            pltpu.make_async_copy(eden.at[pl.ds(ebase0 + c * CHUNK, CHUNK)],
                                  edenb.at[s], dsems[s]).wait()

    def _drain_out(c, s):
        pltpu.make_async_copy(rows.at[s], shared_acc.at[ridx.at[c]],
                              ssems[s]).wait()
        if do_den:
            pltpu.make_async_copy(edenb.at[s], shared_den.at[ridx.at[c]],
                                  tsems[s]).wait()

    def _process(c, s):
        _wait_in(c, s)
        if do_mul:
            def _mul(i, _2):
                for j in range(DH // 16):
                    sl = pl.ds(j * 16, 16)
                    rows[s, i, sl] = rows[s, i, sl] * ewb[s, i, sl]
                return 0

            lax.fori_loop(0, CHUNK, _mul, 0)
        pltpu.async_copy(rows.at[s], shared_acc.at[ridx.at[c]], ssems[s],
                         add=True)
        if do_den:
            pltpu.async_copy(edenb.at[s], shared_den.at[ridx.at[c]], tsems[s],
                             add=True)

    _issue(0, 0)
    _issue(1, 1)

    def _chunk3(c3, _):
        for s in range(3):
            c = c3 * 3 + s
            prv = (s + 2) % 3

            _process(c, s)

            @pl.when(c >= 1)
            def _dr():
                _drain_out(c - 1, prv)

            @pl.when(c < CHUNKS_PER_TILE - 2)
            def _pref():
                _issue(c + 2, prv)
        return 0

    lax.fori_loop(0, CHUNKS_PER_TILE // 3, _chunk3, 0)

    c_last = CHUNKS_PER_TILE - 1
    _process(c_last, c_last % 3)
    _drain_out(c_last - 1, (c_last - 1) % 3)
    _drain_out(c_last, c_last % 3)


def _copy_out(shared_acc, out, sid):
    @pl.when(sid < NSUB - 1)
    def _full():
        osl = pl.ds(sid * ROWS_PER_TILE, ROWS_PER_TILE)
        pltpu.sync_copy(shared_acc.at[osl], out.at[osl])

    @pl.when(sid == NSUB - 1)
    def _part():
        osl = pl.ds((NSUB - 1) * ROWS_PER_TILE, LAST_ROWS)
        pltpu.sync_copy(shared_acc.at[osl], out.at[osl])


def _sc_body(s3d, r3d, xlo, xhi, vlo, vhi, ewlo, ewhi, eden,
             msg_lo, msg_hi, mag_lo, mag_hi, den_out,
             sidx, ridx, rows, ewb, edenb,
             shared_acc, shared_den,
             gsem0, gsem1, gsem2, esem0, esem1, esem2,
             dsem0, dsem1, dsem2, ssem0, ssem1, ssem2,
             tsem0, tsem1, tsem2):
    gsems = (gsem0, gsem1, gsem2)
    esems = (esem0, esem1, esem2)
    dsems = (dsem0, dsem1, dsem2)
    ssems = (ssem0, ssem1, ssem2)
    tsems = (tsem0, tsem1, tsem2)
    cid = lax.axis_index("c")
    sid = lax.axis_index("s")

    pltpu.sync_copy(s3d.at[sid], sidx)
    pltpu.sync_copy(r3d.at[sid], ridx)

    _zero_rows(rows)
    zero16 = jnp.zeros((16,), jnp.float32)

    def _zd(i, _):
        edenb[0, i, pl.ds(0, 16)] = zero16
        return 0

    lax.fori_loop(0, CHUNK, _zd, 0)
    _zero_acc_slice(rows, shared_acc, sid)
    base = sid * ROWS_PER_TILE
    for z in range(ROWS_PER_TILE // CHUNK):
        pltpu.sync_copy(edenb.at[0], shared_den.at[pl.ds(base + z * CHUNK, CHUNK)])
    plsc.subcore_barrier()

    @pl.when(cid == 0)
    def _a0():
        _gather_pass(xlo, ewlo, eden, sidx, ridx, rows, ewb, edenb,
                     shared_acc, shared_den, gsems, esems, dsems, ssems, tsems, sid, True, True)

    @pl.when(cid == 1)
    def _a1():
        _gather_pass(xhi, ewhi, eden, sidx, ridx, rows, ewb, edenb,
                     shared_acc, shared_den, gsems, esems, dsems, ssems, tsems, sid, True, False)

    plsc.subcore_barrier()

    @pl.when(cid == 0)
    def _oa0():
        _copy_out(shared_acc, msg_lo, sid)
        dsl = pl.ds(sid * ROWS_PER_TILE, ROWS_PER_TILE)
        pltpu.sync_copy(shared_den.at[dsl], den_out.at[dsl])

    @pl.when(cid == 1)
    def _oa1():
        _copy_out(shared_acc, msg_hi, sid)

    plsc.subcore_barrier()

    _zero_rows(rows)
    _zero_acc_slice(rows, shared_acc, sid)
    plsc.subcore_barrier()

    @pl.when(cid == 0)
    def _b0():
        _gather_pass(vlo, ewlo, eden, sidx, ridx, rows, ewb, edenb,
                     shared_acc, shared_den, gsems, esems, dsems, ssems, tsems, sid, False, False)

    @pl.when(cid == 1)
    def _b1():
        _gather_pass(vhi, ewhi, eden, sidx, ridx, rows, ewb, edenb,
                     shared_acc, shared_den, gsems, esems, dsems, ssems, tsems, sid, False, False)

    plsc.subcore_barrier()

    @pl.when(cid == 0)
    def _ob0():
        _copy_out(shared_acc, mag_lo, sid)

    @pl.when(cid == 1)
    def _ob1():
        _copy_out(shared_acc, mag_hi, sid)


def _sc_call(s3d, r3d, xlo, xhi, vlo, vhi, ewlo, ewhi, eden):
    mesh = plsc.VectorSubcoreMesh(core_axis_name="c", subcore_axis_name="s")
    f = pl.kernel(
        _sc_body,
        compiler_params=pltpu.CompilerParams(use_tc_tiling_on_sc=False),
        out_type=[
            jax.ShapeDtypeStruct((N, DH), jnp.float32),
            jax.ShapeDtypeStruct((N, DH), jnp.float32),
            jax.ShapeDtypeStruct((N, DH), jnp.float32),
            jax.ShapeDtypeStruct((N, DH), jnp.float32),
            jax.ShapeDtypeStruct((NPAD, DEN_W), jnp.float32),
        ],
        mesh=mesh,
        scratch_types=[
            pltpu.VMEM((CHUNKS_PER_TILE, CHUNK), jnp.int32),
            pltpu.VMEM((CHUNKS_PER_TILE, CHUNK), jnp.int32),
            pltpu.VMEM((3, CHUNK, DH), jnp.float32),
            pltpu.VMEM((3, CHUNK, DH), jnp.float32),
            pltpu.VMEM((3, CHUNK, DEN_W), jnp.float32),
            pltpu.VMEM_SHARED((NPAD, DH), jnp.float32),
            pltpu.VMEM_SHARED((NPAD, DEN_W), jnp.float32),
            pltpu.SemaphoreType.DMA,
            pltpu.SemaphoreType.DMA,
            pltpu.SemaphoreType.DMA,
            pltpu.SemaphoreType.DMA,
            pltpu.SemaphoreType.DMA,
            pltpu.SemaphoreType.DMA,
            pltpu.SemaphoreType.DMA,
            pltpu.SemaphoreType.DMA,
            pltpu.SemaphoreType.DMA,
            pltpu.SemaphoreType.DMA,
            pltpu.SemaphoreType.DMA,
            pltpu.SemaphoreType.DMA,
            pltpu.SemaphoreType.DMA,
            pltpu.SemaphoreType.DMA,
            pltpu.SemaphoreType.DMA,
        ],
    )
    return f(s3d, r3d, xlo, xhi, vlo, vhi, ewlo, ewhi, eden)


def _post_body(mlo, mhi, glo, ghi, den, na, wl_lo, wl_hi, wm_lo, wm_hi,
               wskip, wmagskip, o1, o2):
    m = (jnp.dot(mlo[...], wl_lo[...], preferred_element_type=jnp.float32)
         + jnp.dot(mhi[...], wl_hi[...], preferred_element_type=jnp.float32)) * _INV_SQRT_D
    m = m / (den[...] + 1.0)
    g = (jnp.dot(glo[...], wm_lo[...], preferred_element_type=jnp.float32)
         + jnp.dot(ghi[...], wm_hi[...], preferred_element_type=jnp.float32)) * (
        _INV_SQRT_D / AVG_NEIGH)
    a = na[...]
    acc1 = jnp.zeros((_NB, D), jnp.float32)
    acc2 = jnp.zeros((_NB, D), jnp.float32)
    for vv in range(NELEM):
        av = a[:, vv:vv + 1]
        acc1 = acc1 + jnp.dot(m * av, wskip[vv], preferred_element_type=jnp.float32)
        acc2 = acc2 + jnp.dot(g * av, wmagskip[vv], preferred_element_type=jnp.float32)
    o1[...] = acc1 * _INV_SQRT_UV
    o2[...] = acc2 * _INV_SQRT_UV


def _post_call(mlo, mhi, glo, ghi, den2d, node_attrs,
               wl_lo, wl_hi, wm_lo, wm_hi, W_skip_t, W_magskip_t):
    grid = (N // _NB,)
    return pl.pallas_call(
        _post_body,
        grid=grid,
        in_specs=[
            pl.BlockSpec((_NB, DH), lambda i: (i, 0)),
            pl.BlockSpec((_NB, DH), lambda i: (i, 0)),
            pl.BlockSpec((_NB, DH), lambda i: (i, 0)),
            pl.BlockSpec((_NB, DH), lambda i: (i, 0)),
            pl.BlockSpec((_NB, 1), lambda i: (i, 0)),
            pl.BlockSpec((_NB, NELEM), lambda i: (i, 0)),
            pl.BlockSpec((DH, D), lambda i: (0, 0)),
            pl.BlockSpec((DH, D), lambda i: (0, 0)),
            pl.BlockSpec((DH, D), lambda i: (0, 0)),
            pl.BlockSpec((DH, D), lambda i: (0, 0)),
            pl.BlockSpec((NELEM, D, D), lambda i: (0, 0, 0)),
            pl.BlockSpec((NELEM, D, D), lambda i: (0, 0, 0)),
        ],
        out_specs=[
            pl.BlockSpec((_NB, D), lambda i: (i, 0)),
            pl.BlockSpec((_NB, D), lambda i: (i, 0)),
        ],
        out_shape=[
            jax.ShapeDtypeStruct((N, D), jnp.float32),
            jax.ShapeDtypeStruct((N, D), jnp.float32),
        ],
    )(mlo, mhi, glo, ghi, den2d, node_attrs, wl_lo, wl_hi, wm_lo, wm_hi,
      W_skip_t, W_magskip_t)


def kernel(node_attrs, node_feats, edge_attrs, edge_feats, edge_index,
           magmom_node_inv_feats, magmom_node_attrs,
           W_up, Wr1, Wr2, Wr3, Wr4, W_density, W_mag_radial,
           W_lin, W_maglin, W_skip, W_magskip):
    xlo, xhi, vlo, vhi = _pre_call(node_feats, magmom_node_inv_feats,
                                   magmom_node_attrs, W_up, W_mag_radial)
    ewlo, ewhi, eden = _edge_call(edge_feats, edge_attrs, Wr1, Wr2, Wr3, Wr4,
                                  W_density)
    s3d = edge_index[0].reshape(NSUB, CHUNKS_PER_TILE, CHUNK)
    r3d = edge_index[1].reshape(NSUB, CHUNKS_PER_TILE, CHUNK)
    msg_lo, msg_hi, mag_lo, mag_hi, den = _sc_call(
        s3d, r3d, xlo, xhi, vlo, vhi, ewlo, ewhi, eden)
    W_skip_t = jnp.transpose(W_skip, (1, 0, 2))
    W_magskip_t = jnp.transpose(W_magskip, (1, 0, 2))
    den1 = den[:N, 0:1]
    o1, o2 = _post_call(msg_lo, msg_hi, mag_lo, mag_hi, den1,
                        node_attrs, W_lin[:DH], W_lin[DH:],
                        W_maglin[:DH], W_maglin[DH:], W_skip_t, W_magskip_t)
    return (o1.reshape(N, D, 1), o2.reshape(N, D, 1))


# speedup vs baseline: 4.9651x; 1.2838x over previous
import math

import jax
import jax.numpy as jnp
from jax import lax
from jax.experimental import pallas as pl
from jax.experimental.pallas import tpu as pltpu
from jax.experimental.pallas import tpu_sc as plsc

N = 10000
E = 320000
D = 128
DH = D // 2
NELEM = 10
RADIAL_IN = 8
MAG_IN = 16
H = 64
AVG_NEIGH = 32.0

NSUB = 16
EDGES_PER_TILE = E // NSUB
CHUNK = 80
CHUNKS_PER_TILE = EDGES_PER_TILE // CHUNK
NPAD = 10240
ROWS_PER_TILE = NPAD // NSUB
LAST_ROWS = N - (NSUB - 1) * ROWS_PER_TILE
DEN_W = 16
DEN_ROWS = NPAD // 16
DEN_ROWS_PER_TILE = DEN_ROWS // NSUB

_INV_SQRT_D = 1.0 / math.sqrt(float(D))
_INV_SQRT_R = 1.0 / math.sqrt(float(RADIAL_IN))
_INV_SQRT_H = 1.0 / math.sqrt(float(H))
_INV_SQRT_M = 1.0 / math.sqrt(float(MAG_IN))
_INV_SQRT_UV = 1.0 / math.sqrt(float(D * NELEM))


_NB = 1000


def _pre_body(nf, minv, mga, wup, wmr, xlo, xhi, vlo, vhi):
    x = jnp.dot(nf[...], wup[...], preferred_element_type=jnp.float32) * _INV_SQRT_D
    magr = jnp.dot(minv[...], wmr[...], preferred_element_type=jnp.float32) * _INV_SQRT_M
    v = x * mga[...] * magr
    xlo[...] = x[:, :DH]
    xhi[...] = x[:, DH:]
    vlo[...] = v[:, :DH]
    vhi[...] = v[:, DH:]


def _pre_call(node_feats, minv, mga, W_up, W_mag_radial):
    grid = (N // _NB,)
    return pl.pallas_call(
        _pre_body,
        grid=grid,
        in_specs=[
            pl.BlockSpec((_NB, D), lambda i: (i, 0)),
            pl.BlockSpec((_NB, MAG_IN), lambda i: (i, 0)),
            pl.BlockSpec((_NB, 1), lambda i: (i, 0)),
            pl.BlockSpec((D, D), lambda i: (0, 0)),
            pl.BlockSpec((MAG_IN, D), lambda i: (0, 0)),
        ],
        out_specs=[pl.BlockSpec((_NB, DH), lambda i: (i, 0))] * 4,
        out_shape=[jax.ShapeDtypeStruct((N, DH), jnp.float32)] * 4,
    )(node_feats, minv, mga, W_up, W_mag_radial)


_EB = 2000


def _edge_body(ef, ea, wr1, wr2, wr3, wr4, wd, ew_out, eden_out):
    bf = jnp.bfloat16
    f = ef[...]
    h = jax.nn.silu(jnp.dot(f, wr1[...],
                            preferred_element_type=jnp.float32) * _INV_SQRT_R)
    h = jax.nn.silu(jnp.dot(h.astype(bf), wr2[...].astype(bf),
                            preferred_element_type=jnp.float32) * _INV_SQRT_H)
    h = jax.nn.silu(jnp.dot(h.astype(bf), wr3[...].astype(bf),
                            preferred_element_type=jnp.float32) * _INV_SQRT_H)
    tpw = jnp.dot(h.astype(bf), wr4[...].astype(bf),
                  preferred_element_type=jnp.float32) * _INV_SQRT_H
    ew_out[...] = tpw * ea[...]
    dd = jnp.dot(f, wd[...], preferred_element_type=jnp.float32) * _INV_SQRT_R
    d = jnp.tanh(dd * dd)
    lane = lax.broadcasted_iota(jnp.int32, (_EB, DEN_W), 1)
    eden_out[...] = jnp.where(lane == 0, d, 0.0)


def _edge_call(edge_feats, edge_attrs, Wr1, Wr2, Wr3, Wr4, W_density):
    grid = (E // _EB,)
    return pl.pallas_call(
        _edge_body,
        grid=grid,
        in_specs=[
            pl.BlockSpec((_EB, RADIAL_IN), lambda i: (i, 0)),
            pl.BlockSpec((_EB, 1), lambda i: (i, 0)),
            pl.BlockSpec((RADIAL_IN, H), lambda i: (0, 0)),
            pl.BlockSpec((H, H), lambda i: (0, 0)),
            pl.BlockSpec((H, H), lambda i: (0, 0)),
            pl.BlockSpec((H, D), lambda i: (0, 0)),
            pl.BlockSpec((RADIAL_IN, 1), lambda i: (0, 0)),
        ],
        out_specs=[
            pl.BlockSpec((_EB, D), lambda i: (i, 0)),
            pl.BlockSpec((_EB, DEN_W), lambda i: (i, 0)),
        ],
        out_shape=[
            jax.ShapeDtypeStruct((E, D), jnp.float32),
            jax.ShapeDtypeStruct((E, DEN_W), jnp.float32),
        ],
    )(edge_feats, edge_attrs, Wr1, Wr2, Wr3, Wr4, W_density)


def _zero_rows(rows):
    zero16 = jnp.zeros((16,), jnp.float32)

    def _z(i, _):
        for j in range(DH // 16):
            rows[0, i, pl.ds(j * 16, 16)] = zero16
        return 0

    lax.fori_loop(0, CHUNK, _z, 0)


def _zero_acc_slice(rows, shared_acc, sid):
    base = sid * ROWS_PER_TILE
    for z in range(ROWS_PER_TILE // CHUNK):
        pltpu.sync_copy(rows.at[0], shared_acc.at[pl.ds(base + z * CHUNK, CHUNK)])


def _gather_pass(src, ewsrc, eden, sidx, ridx, rows, ewb, edenb,
                 shared_acc, shared_den, gsems, esems, dsems, ssems, tsems,
                 sid, ecol, do_mul, do_den):
    ebase0 = sid * EDGES_PER_TILE

    def _issue(c, s):
        pltpu.async_copy(src.at[sidx.at[c]], rows.at[s], gsems[s])
        if do_mul:
            pltpu.async_copy(
                ewsrc.at[pl.ds(ebase0 + c * CHUNK, CHUNK), pl.ds(ecol, DH)],
                ewb.at[s], esems[s])
        if do_den:
            pltpu.async_copy(eden.at[pl.ds(ebase0 + c * CHUNK, CHUNK)],
                             edenb.at[s], dsems[s])

    def _wait_in(c, s):
        pltpu.make_async_copy(src.at[sidx.at[c]], rows.at[s], gsems[s]).wait()
        if do_mul:
            pltpu.make_async_copy(
                ewsrc.at[pl.ds(ebase0 + c * CHUNK, CHUNK), pl.ds(ecol, DH)],
                ewb.at[s], esems[s]).wait()
        if do_den:
            pltpu.make_async_copy(eden.at[pl.ds(ebase0 + c * CHUNK, CHUNK)],
                                  edenb.at[s], dsems[s]).wait()

    def _drain_out(c, s):
        pltpu.make_async_copy(rows.at[s], shared_acc.at[ridx.at[c]],
                              ssems[s]).wait()
        if do_den:
            pltpu.make_async_copy(edenb.at[s], shared_den.at[ridx.at[c]],
                                  tsems[s]).wait()

    def _process(c, s):
        _wait_in(c, s)
        if do_mul:
            def _mul(i, _2):
                for j in range(DH // 16):
                    sl = pl.ds(j * 16, 16)
                    rows[s, i, sl] = rows[s, i, sl] * ewb[s, i, sl]
                return 0

            lax.fori_loop(0, CHUNK, _mul, 0)
        pltpu.async_copy(rows.at[s], shared_acc.at[ridx.at[c]], ssems[s],
                         add=True)
        if do_den:
            pltpu.async_copy(edenb.at[s], shared_den.at[ridx.at[c]], tsems[s],
                             add=True)

    _issue(0, 0)
    _issue(1, 1)

    def _chunk3(c3, _):
        for s in range(3):
            c = c3 * 3 + s
            prv = (s + 2) % 3

            _process(c, s)

            @pl.when(c >= 1)
            def _dr():
                _drain_out(c - 1, prv)

            @pl.when(c < CHUNKS_PER_TILE - 2)
            def _pref():
                _issue(c + 2, prv)
        return 0

    lax.fori_loop(0, CHUNKS_PER_TILE // 3, _chunk3, 0)

    c_last = CHUNKS_PER_TILE - 1
    _process(c_last, c_last % 3)
    _drain_out(c_last - 1, (c_last - 1) % 3)
    _drain_out(c_last, c_last % 3)


def _copy_out(shared_acc, out, sid):
    @pl.when(sid < NSUB - 1)
    def _full():
        osl = pl.ds(sid * ROWS_PER_TILE, ROWS_PER_TILE)
        pltpu.sync_copy(shared_acc.at[osl], out.at[osl])

    @pl.when(sid == NSUB - 1)
    def _part():
        osl = pl.ds((NSUB - 1) * ROWS_PER_TILE, LAST_ROWS)
        pltpu.sync_copy(shared_acc.at[osl], out.at[osl])


def _sc_body(s3d, r3d, xlo, xhi, vlo, vhi, ew, eden,
             msg_lo, msg_hi, mag_lo, mag_hi, den_out,
             sidx, ridx, rows, ewb, edenb,
             shared_acc, shared_den,
             gsem0, gsem1, gsem2, esem0, esem1, esem2,
             dsem0, dsem1, dsem2, ssem0, ssem1, ssem2,
             tsem0, tsem1, tsem2):
    gsems = (gsem0, gsem1, gsem2)
    esems = (esem0, esem1, esem2)
    dsems = (dsem0, dsem1, dsem2)
    ssems = (ssem0, ssem1, ssem2)
    tsems = (tsem0, tsem1, tsem2)
    cid = lax.axis_index("c")
    sid = lax.axis_index("s")

    pltpu.sync_copy(s3d.at[sid], sidx)
    pltpu.sync_copy(r3d.at[sid], ridx)

    _zero_rows(rows)
    zero16 = jnp.zeros((16,), jnp.float32)

    def _zd(i, _):
        edenb[0, i, pl.ds(0, 16)] = zero16
        return 0

    lax.fori_loop(0, CHUNK, _zd, 0)
    _zero_acc_slice(rows, shared_acc, sid)
    base = sid * ROWS_PER_TILE
    for z in range(ROWS_PER_TILE // CHUNK):
        pltpu.sync_copy(edenb.at[0], shared_den.at[pl.ds(base + z * CHUNK, CHUNK)])
    plsc.subcore_barrier()

    @pl.when(cid == 0)
    def _a0():
        _gather_pass(xlo, ew, eden, sidx, ridx, rows, ewb, edenb,
                     shared_acc, shared_den, gsems, esems, dsems, ssems,
                     tsems, sid, 0, True, True)

    @pl.when(cid == 1)
    def _a1():
        _gather_pass(xhi, ew, eden, sidx, ridx, rows, ewb, edenb,
                     shared_acc, shared_den, gsems, esems, dsems, ssems,
                     tsems, sid, DH, True, False)

    plsc.subcore_barrier()

    @pl.when(cid == 0)
    def _oa0():
        _copy_out(shared_acc, msg_lo, sid)
        dsl = pl.ds(sid * ROWS_PER_TILE, ROWS_PER_TILE)
        pltpu.sync_copy(shared_den.at[dsl], den_out.at[dsl])

    @pl.when(cid == 1)
    def _oa1():
        _copy_out(shared_acc, msg_hi, sid)

    plsc.subcore_barrier()

    _zero_rows(rows)
    _zero_acc_slice(rows, shared_acc, sid)
    plsc.subcore_barrier()

    @pl.when(cid == 0)
    def _b0():
        _gather_pass(vlo, ew, eden, sidx, ridx, rows, ewb, edenb,
                     shared_acc, shared_den, gsems, esems, dsems, ssems,
                     tsems, sid, 0, False, False)

    @pl.when(cid == 1)
    def _b1():
        _gather_pass(vhi, ew, eden, sidx, ridx, rows, ewb, edenb,
                     shared_acc, shared_den, gsems, esems, dsems, ssems,
                     tsems, sid, 0, False, False)

    plsc.subcore_barrier()

    @pl.when(cid == 0)
    def _ob0():
        _copy_out(shared_acc, mag_lo, sid)

    @pl.when(cid == 1)
    def _ob1():
        _copy_out(shared_acc, mag_hi, sid)


def _sc_call(s3d, r3d, xlo, xhi, vlo, vhi, ew, eden):
    mesh = plsc.VectorSubcoreMesh(core_axis_name="c", subcore_axis_name="s")
    f = pl.kernel(
        _sc_body,
        compiler_params=pltpu.CompilerParams(use_tc_tiling_on_sc=False),
        out_type=[
            jax.ShapeDtypeStruct((N, DH), jnp.float32),
            jax.ShapeDtypeStruct((N, DH), jnp.float32),
            jax.ShapeDtypeStruct((N, DH), jnp.float32),
            jax.ShapeDtypeStruct((N, DH), jnp.float32),
            jax.ShapeDtypeStruct((NPAD, DEN_W), jnp.float32),
        ],
        mesh=mesh,
        scratch_types=[
            pltpu.VMEM((CHUNKS_PER_TILE, CHUNK), jnp.int32),
            pltpu.VMEM((CHUNKS_PER_TILE, CHUNK), jnp.int32),
            pltpu.VMEM((3, CHUNK, DH), jnp.float32),
            pltpu.VMEM((3, CHUNK, DH), jnp.float32),
            pltpu.VMEM((3, CHUNK, DEN_W), jnp.float32),
            pltpu.VMEM_SHARED((NPAD, DH), jnp.float32),
            pltpu.VMEM_SHARED((NPAD, DEN_W), jnp.float32),
            pltpu.SemaphoreType.DMA,
            pltpu.SemaphoreType.DMA,
            pltpu.SemaphoreType.DMA,
            pltpu.SemaphoreType.DMA,
            pltpu.SemaphoreType.DMA,
            pltpu.SemaphoreType.DMA,
            pltpu.SemaphoreType.DMA,
            pltpu.SemaphoreType.DMA,
            pltpu.SemaphoreType.DMA,
            pltpu.SemaphoreType.DMA,
            pltpu.SemaphoreType.DMA,
            pltpu.SemaphoreType.DMA,
            pltpu.SemaphoreType.DMA,
            pltpu.SemaphoreType.DMA,
            pltpu.SemaphoreType.DMA,
        ],
    )
    return f(s3d, r3d, xlo, xhi, vlo, vhi, ew, eden)


def _post_body(mlo, mhi, glo, ghi, den, na, wl_lo, wl_hi, wm_lo, wm_hi,
               wskip, wmagskip, o1, o2):
    m = (jnp.dot(mlo[...], wl_lo[...], preferred_element_type=jnp.float32)
         + jnp.dot(mhi[...], wl_hi[...], preferred_element_type=jnp.float32)) * _INV_SQRT_D
    m = m / (den[...] + 1.0)
    g = (jnp.dot(glo[...], wm_lo[...], preferred_element_type=jnp.float32)
         + jnp.dot(ghi[...], wm_hi[...], preferred_element_type=jnp.float32)) * (
        _INV_SQRT_D / AVG_NEIGH)
    a = na[...]
    acc1 = jnp.zeros((_NB, D), jnp.float32)
    acc2 = jnp.zeros((_NB, D), jnp.float32)
    for vv in range(NELEM):
        av = a[:, vv:vv + 1]
        acc1 = acc1 + jnp.dot(m * av, wskip[vv], preferred_element_type=jnp.float32)
        acc2 = acc2 + jnp.dot(g * av, wmagskip[vv], preferred_element_type=jnp.float32)
    o1[...] = acc1 * _INV_SQRT_UV
    o2[...] = acc2 * _INV_SQRT_UV


def _post_call(mlo, mhi, glo, ghi, den2d, node_attrs,
               wl_lo, wl_hi, wm_lo, wm_hi, W_skip_t, W_magskip_t):
    grid = (N // _NB,)
    return pl.pallas_call(
        _post_body,
        grid=grid,
        in_specs=[
            pl.BlockSpec((_NB, DH), lambda i: (i, 0)),
            pl.BlockSpec((_NB, DH), lambda i: (i, 0)),
            pl.BlockSpec((_NB, DH), lambda i: (i, 0)),
            pl.BlockSpec((_NB, DH), lambda i: (i, 0)),
            pl.BlockSpec((_NB, 1), lambda i: (i, 0)),
            pl.BlockSpec((_NB, NELEM), lambda i: (i, 0)),
            pl.BlockSpec((DH, D), lambda i: (0, 0)),
            pl.BlockSpec((DH, D), lambda i: (0, 0)),
            pl.BlockSpec((DH, D), lambda i: (0, 0)),
            pl.BlockSpec((DH, D), lambda i: (0, 0)),
            pl.BlockSpec((NELEM, D, D), lambda i: (0, 0, 0)),
            pl.BlockSpec((NELEM, D, D), lambda i: (0, 0, 0)),
        ],
        out_specs=[
            pl.BlockSpec((_NB, D), lambda i: (i, 0)),
            pl.BlockSpec((_NB, D), lambda i: (i, 0)),
        ],
        out_shape=[
            jax.ShapeDtypeStruct((N, D), jnp.float32),
            jax.ShapeDtypeStruct((N, D), jnp.float32),
        ],
    )(mlo, mhi, glo, ghi, den2d, node_attrs, wl_lo, wl_hi, wm_lo, wm_hi,
      W_skip_t, W_magskip_t)


def kernel(node_attrs, node_feats, edge_attrs, edge_feats, edge_index,
           magmom_node_inv_feats, magmom_node_attrs,
           W_up, Wr1, Wr2, Wr3, Wr4, W_density, W_mag_radial,
           W_lin, W_maglin, W_skip, W_magskip):
    xlo, xhi, vlo, vhi = _pre_call(node_feats, magmom_node_inv_feats,
                                   magmom_node_attrs, W_up, W_mag_radial)
    ew, eden = _edge_call(edge_feats, edge_attrs, Wr1, Wr2, Wr3, Wr4,
                          W_density)
    s3d = edge_index[0].reshape(NSUB, CHUNKS_PER_TILE, CHUNK)
    r3d = edge_index[1].reshape(NSUB, CHUNKS_PER_TILE, CHUNK)
    msg_lo, msg_hi, mag_lo, mag_hi, den = _sc_call(
        s3d, r3d, xlo, xhi, vlo, vhi, ew, eden)
    W_skip_t = jnp.transpose(W_skip, (1, 0, 2))
    W_magskip_t = jnp.transpose(W_magskip, (1, 0, 2))
    den1 = den[:N, 0:1]
    o1, o2 = _post_call(msg_lo, msg_hi, mag_lo, mag_hi, den1,
                        node_attrs, W_lin[:DH], W_lin[DH:],
                        W_maglin[:DH], W_maglin[DH:], W_skip_t, W_magskip_t)
    return (o1.reshape(N, D, 1), o2.reshape(N, D, 1))
